# baseline (speedup 1.0000x reference)
import functools

import jax
import jax.numpy as jnp
import numpy as np
from jax.experimental import pallas as pl
from jax.experimental.pallas import tpu as pltpu
from jax.sharding import Mesh, PartitionSpec as P

D = 768
H = 768
E = 8
O = 3
N = 2048


def _router_kernel(x_ref, pW1_ref, pb1_ref, pW2_ref, pb2_ref, probs_ref):
    x = x_ref[...]
    h = jnp.maximum(
        jnp.dot(x, pW1_ref[...], preferred_element_type=jnp.float32)
        + pb1_ref[...], 0.0)
    logits = jnp.dot(h, pW2_ref[...],
                     preferred_element_type=jnp.float32) + pb2_ref[...]
    m = jnp.max(logits, axis=-1, keepdims=True)
    ex = jnp.exp(logits - m)
    probs_ref[...] = ex / jnp.sum(ex, axis=-1, keepdims=True)


def _run_router(x, pW1, pb1, pW2, pb2):
    return pl.pallas_call(
        _router_kernel,
        out_shape=jax.ShapeDtypeStruct((N, E + 1), jnp.float32),
    )(x, pW1, pb1.reshape(1, H), pW2, pb2.reshape(1, E + 1))


def _expert_kernel(el, x_ref, probs_ref, pid_ref,
                   projW_ref, projb_ref, rl_ref,
                   W1_ref, b1_ref, W2_ref, b2_ref,
                   out_ref):
    e = pl.program_id(0)
    xb = x_ref[...]

    xp = jnp.dot(xb, projW_ref[0], preferred_element_type=jnp.float32)
    xp = xp + projb_ref[0]

    rl = rl_ref[0]
    rm = jnp.max(rl, axis=-1, keepdims=True)
    re_ = jnp.exp(rl - rm)
    w = re_ / jnp.sum(re_, axis=-1, keepdims=True)

    pr = probs_ref[...]
    lane = jax.lax.broadcasted_iota(jnp.int32, (N, el), 1)
    pcol = jnp.sum(jnp.where(lane == e, pr, 0.0), axis=1, keepdims=True)

    @pl.when(e == 0)
    def _init():
        out_ref[...] = pid_ref[...] * xb

    b1 = b1_ref[0]
    b2 = b2_ref[0]
    for o in range(O):
        h1 = jnp.maximum(
            jnp.dot(xp, W1_ref[0, o], preferred_element_type=jnp.float32)
            + b1[o:o + 1], 0.0)
        h2 = jnp.maximum(
            jnp.dot(h1, W2_ref[0, o], preferred_element_type=jnp.float32)
            + b2[o:o + 1], 0.0)
        out_ref[...] += (pcol * w[:, o:o + 1]) * h2


def _run_experts(el, x, probs_local, pid, proj_W, proj_b, route_logits,
                 op_W1, op_b1, op_W2, op_b2):
    return pl.pallas_call(
        functools.partial(_expert_kernel, el),
        grid=(el,),
        in_specs=[
            pl.BlockSpec((N, D), lambda e: (0, 0)),
            pl.BlockSpec((N, el), lambda e: (0, 0)),
            pl.BlockSpec((N, 1), lambda e: (0, 0)),
            pl.BlockSpec((1, D, H), lambda e: (e, 0, 0)),
            pl.BlockSpec((1, 1, H), lambda e: (e, 0, 0)),
            pl.BlockSpec((1, 1, O), lambda e: (e, 0, 0)),
            pl.BlockSpec((1, O, H, H), lambda e: (e, 0, 0, 0)),
            pl.BlockSpec((1, O, H), lambda e: (e, 0, 0)),
            pl.BlockSpec((1, O, H, H), lambda e: (e, 0, 0, 0)),
            pl.BlockSpec((1, O, H), lambda e: (e, 0, 0)),
        ],
        out_specs=pl.BlockSpec((N, H), lambda e: (0, 0)),
        out_shape=jax.ShapeDtypeStruct((N, H), jnp.float32),
        compiler_params=pltpu.CompilerParams(
            dimension_semantics=("arbitrary",),
            vmem_limit_bytes=63 * 1024 * 1024,
        ),
    )(x, probs_local, pid, proj_W, proj_b.reshape(el, 1, H),
      route_logits.reshape(el, 1, O), op_W1, op_b1, op_W2, op_b2)


def _single_device(x, proj_W, proj_b, route_logits, op_W1, op_b1,
                   op_W2, op_b2, pW1, pb1, pW2, pb2):
    probs = _run_router(x, pW1, pb1, pW2, pb2)
    pid = probs[:, E:E + 1] * jnp.ones((N, 1), jnp.float32)
    return _run_experts(E, x, probs[:, :E], pid, proj_W, proj_b,
                        route_logits, op_W1, op_b1, op_W2, op_b2)


def _sharded_body(x, proj_W, proj_b, route_logits, op_W1, op_b1,
                  op_W2, op_b2, pW1, pb1, pW2, pb2):
    el = proj_W.shape[0]
    probs = _run_router(x, pW1, pb1, pW2, pb2)
    idx = jax.lax.axis_index("x")
    probs_local = jax.lax.dynamic_slice(probs, (0, idx * el), (N, el))
    id_scale = jnp.where(idx == 0, 1.0, 0.0)
    pid = probs[:, E:E + 1] * id_scale
    partial = _run_experts(el, x, probs_local, pid, proj_W, proj_b,
                           route_logits, op_W1, op_b1, op_W2, op_b2)
    return jax.lax.psum(partial, "x")


def kernel(x, proj_W, proj_b, route_logits, op_W1, op_b1, op_W2, op_b2,
           pW1, pb1, pW2, pb2):
    devs = jax.devices()
    if len(devs) >= 2 and E % 2 == 0:
        mesh = Mesh(np.array(devs[:2]), ("x",))
        shard = P("x")
        rep = P()
        fn = jax.shard_map(
            _sharded_body,
            mesh=mesh,
            in_specs=(rep, shard, shard, shard, shard, shard, shard, shard,
                      rep, rep, rep, rep),
            out_specs=rep,
            check_vma=False,
        )
        return fn(x, proj_W, proj_b, route_logits, op_W1, op_b1,
                  op_W2, op_b2, pW1, pb1, pW2, pb2)
    return _single_device(x, proj_W, proj_b, route_logits, op_W1, op_b1,
                          op_W2, op_b2, pW1, pb1, pW2, pb2)


# speedup vs baseline: 2.6139x; 2.6139x over previous
import jax
import jax.numpy as jnp
from jax.experimental import pallas as pl
from jax.experimental.pallas import tpu as pltpu

D = 768
H = 768
E = 8
O = 3
N = 2048


def _router_kernel(x_ref, pW1_ref, pb1_ref, pW2_ref, pb2_ref, probs_ref):
    x = x_ref[...]
    h = jnp.maximum(
        jnp.dot(x, pW1_ref[...], preferred_element_type=jnp.float32)
        + pb1_ref[...], 0.0)
    logits = jnp.dot(h, pW2_ref[...],
                     preferred_element_type=jnp.float32) + pb2_ref[...]
    m = jnp.max(logits, axis=-1, keepdims=True)
    ex = jnp.exp(logits - m)
    probs_ref[...] = ex / jnp.sum(ex, axis=-1, keepdims=True)


def _expert_kernel(x_ref, probs_ref,
                   projW_ref, projb_ref, rl_ref,
                   W1_ref, b1_ref, W2_ref, b2_ref,
                   out_ref):
    e = pl.program_id(0)
    xb = x_ref[...]

    xp = jnp.dot(xb, projW_ref[0], preferred_element_type=jnp.float32)
    xp = xp + projb_ref[0]

    rl = rl_ref[0]
    rm = jnp.max(rl, axis=-1, keepdims=True)
    re_ = jnp.exp(rl - rm)
    w = re_ / jnp.sum(re_, axis=-1, keepdims=True)

    pr = probs_ref[...]
    lane = jax.lax.broadcasted_iota(jnp.int32, (N, E + 1), 1)
    pcol = jnp.sum(jnp.where(lane == e, pr, 0.0), axis=1, keepdims=True)

    @pl.when(e == 0)
    def _init():
        pid_col = jnp.sum(jnp.where(lane == E, pr, 0.0),
                          axis=1, keepdims=True)
        out_ref[...] = pid_col * xb

    b1 = b1_ref[0]
    b2 = b2_ref[0]
    for o in range(O):
        h1 = jnp.maximum(
            jnp.dot(xp, W1_ref[0, o], preferred_element_type=jnp.float32)
            + b1[o:o + 1], 0.0)
        h2 = jnp.maximum(
            jnp.dot(h1, W2_ref[0, o], preferred_element_type=jnp.float32)
            + b2[o:o + 1], 0.0)
        out_ref[...] += (pcol * w[:, o:o + 1]) * h2


def kernel(x, proj_W, proj_b, route_logits, op_W1, op_b1, op_W2, op_b2,
           pW1, pb1, pW2, pb2):
    probs = pl.pallas_call(
        _router_kernel,
        out_shape=jax.ShapeDtypeStruct((N, E + 1), jnp.float32),
    )(x, pW1, pb1.reshape(1, H), pW2, pb2.reshape(1, E + 1))

    out = pl.pallas_call(
        _expert_kernel,
        grid=(E,),
        in_specs=[
            pl.BlockSpec((N, D), lambda e: (0, 0)),
            pl.BlockSpec((N, E + 1), lambda e: (0, 0)),
            pl.BlockSpec((1, D, H), lambda e: (e, 0, 0)),
            pl.BlockSpec((1, 1, H), lambda e: (e, 0, 0)),
            pl.BlockSpec((1, 1, O), lambda e: (e, 0, 0)),
            pl.BlockSpec((1, O, H, H), lambda e: (e, 0, 0, 0)),
            pl.BlockSpec((1, O, H), lambda e: (e, 0, 0)),
            pl.BlockSpec((1, O, H, H), lambda e: (e, 0, 0, 0)),
            pl.BlockSpec((1, O, H), lambda e: (e, 0, 0)),
        ],
        out_specs=pl.BlockSpec((N, H), lambda e: (0, 0)),
        out_shape=jax.ShapeDtypeStruct((N, H), jnp.float32),
        compiler_params=pltpu.CompilerParams(
            dimension_semantics=("arbitrary",),
            vmem_limit_bytes=63 * 1024 * 1024,
        ),
    )(
        x,
        probs,
        proj_W,
        proj_b.reshape(E, 1, H),
        route_logits.reshape(E, 1, O),
        op_W1,
        op_b1,
        op_W2,
        op_b2,
    )
    return out
